# Initial kernel scaffold; baseline (speedup 1.0000x reference)
import jax
import jax.numpy as jnp
from jax.experimental import pallas as pl


def kernel(x, tables):
    raise NotImplementedError("write your pallas kernel here")


# speedup vs baseline: 1.3120x; 1.3120x over previous
import functools

import jax
import jax.numpy as jnp
from jax import lax
from jax.experimental import pallas as pl
from jax.experimental.pallas import tpu as pltpu
from jax.experimental.pallas import tpu_sc as plsc

N_CB = 8
CB_SIZE = 2048
D = 1024
B = 4
T = 4096

NUM_CORES = 2
NUM_SUBCORES = 16
NUM_WORKERS = NUM_CORES * NUM_SUBCORES
ROWS_PER_W = (B * T) // NUM_WORKERS
CHUNK = 32
N_CHUNKS = ROWS_PER_W // CHUNK
VREGS_PER_ROW = D // 16


def _body(x_hbm, tab_hbm, out_hbm, idx_v, acc_v, buf0_v, buf1_v,
          sem_a, sem_b0, sem_b1):
    wid = lax.axis_index("s") * NUM_CORES + lax.axis_index("c")
    tpw = T // (NUM_WORKERS // B)
    b = wid // (NUM_WORKERS // B)
    t0 = (wid % (NUM_WORKERS // B)) * tpw
    wbase = wid * ROWS_PER_W

    pltpu.sync_copy(x_hbm.at[b, :, pl.ds(t0, tpw)], idx_v)

    @pl.loop(0, tpw // 16)
    def _offsets(j):
        sl = pl.ds(j * 16, 16)
        for i in range(1, N_CB):
            idx_v[i, sl] = idx_v[i, sl] + i * CB_SIZE

    def _accumulate(buf):
        @pl.loop(0, CHUNK)
        def _rows(r):
            for u in range(VREGS_PER_ROW):
                sl = pl.ds(u * 16, 16)
                plsc.addupdate(acc_v.at[r, sl], buf[r, sl])

    @pl.loop(0, N_CHUNKS)
    def _chunk(ch):
        r0 = ch * CHUNK
        bufs = (buf0_v, buf1_v)
        sems = (sem_b0, sem_b1)

        def gather(i, dst, sem):
            return pltpu.async_copy(
                tab_hbm.at[idx_v.at[i, pl.ds(r0, CHUNK)]], dst, sem)

        g_acc = gather(0, acc_v, sem_a)
        pend = [gather(1, buf0_v, sem_b0), gather(2, buf1_v, sem_b1)]
        g_acc.wait()
        for i in range(1, N_CB):
            k = (i - 1) % 2
            pend[k].wait()
            _accumulate(bufs[k])
            nxt = i + 2
            if nxt < N_CB:
                pend[k] = gather(nxt, bufs[k], sems[k])

        pltpu.sync_copy(acc_v, out_hbm.at[pl.ds(wbase + r0, CHUNK)])


@jax.jit
def _run(x, tables):
    tab_flat = tables.reshape(N_CB * CB_SIZE, D)
    mesh = plsc.VectorSubcoreMesh(core_axis_name="c", subcore_axis_name="s")
    call = pl.kernel(
        _body,
        out_type=jax.ShapeDtypeStruct((B * T, D), jnp.float32),
        mesh=mesh,
        scratch_types=[
            pltpu.VMEM((N_CB, ROWS_PER_W), jnp.int32),
            pltpu.VMEM((CHUNK, D), jnp.float32),
            pltpu.VMEM((CHUNK, D), jnp.float32),
            pltpu.VMEM((CHUNK, D), jnp.float32),
            pltpu.SemaphoreType.DMA,
            pltpu.SemaphoreType.DMA,
            pltpu.SemaphoreType.DMA,
        ],
    )
    out_flat = call(x, tab_flat)
    return out_flat.reshape(B, T, D)


def kernel(x, tables):
    return _run(x.astype(jnp.int32), tables)


# speedup vs baseline: 2.4536x; 1.8702x over previous
import functools

import jax
import jax.numpy as jnp
from jax import lax
from jax.experimental import pallas as pl
from jax.experimental.pallas import tpu as pltpu
from jax.experimental.pallas import tpu_sc as plsc

N_CB = 8
CB_SIZE = 2048
D = 1024
B = 4
T = 4096

NUM_CORES = 2
NUM_SUBCORES = 16
NUM_WORKERS = NUM_CORES * NUM_SUBCORES
ROWS_PER_W = (B * T) // NUM_WORKERS
CHUNK = 32
N_CHUNKS = ROWS_PER_W // CHUNK
VREGS_PER_ROW = D // 16


def _body(x_hbm, tab_hbm, out_hbm, idx_v, acc_v, buf0_v, buf1_v,
          sem_a, sem_b0, sem_b1):
    wid = lax.axis_index("s") * NUM_CORES + lax.axis_index("c")
    tpw = T // (NUM_WORKERS // B)
    b = wid // (NUM_WORKERS // B)
    t0 = (wid % (NUM_WORKERS // B)) * tpw
    wbase = wid * ROWS_PER_W

    pltpu.sync_copy(x_hbm.at[b, :, pl.ds(t0, tpw)], idx_v)

    @pl.loop(0, tpw // 16)
    def _offsets(j):
        sl = pl.ds(j * 16, 16)
        for i in range(1, N_CB):
            idx_v[i, sl] = idx_v[i, sl] + i * CB_SIZE

    def _accumulate(buf):
        @plsc.parallel_loop(0, CHUNK * VREGS_PER_ROW, 1, unroll=16)
        def _pairs(j):
            r = j // VREGS_PER_ROW
            c = (j % VREGS_PER_ROW) * 16
            plsc.addupdate(acc_v.at[r, pl.ds(c, 16)], buf[r, pl.ds(c, 16)])

    @pl.loop(0, N_CHUNKS)
    def _chunk(ch):
        r0 = ch * CHUNK
        bufs = (buf0_v, buf1_v)
        sems = (sem_b0, sem_b1)

        def gather(i, dst, sem):
            return pltpu.async_copy(
                tab_hbm.at[idx_v.at[i, pl.ds(r0, CHUNK)]], dst, sem)

        g_acc = gather(0, acc_v, sem_a)
        pend = [gather(1, buf0_v, sem_b0), gather(2, buf1_v, sem_b1)]
        g_acc.wait()
        for i in range(1, N_CB):
            k = (i - 1) % 2
            pend[k].wait()
            _accumulate(bufs[k])
            nxt = i + 2
            if nxt < N_CB:
                pend[k] = gather(nxt, bufs[k], sems[k])

        pltpu.sync_copy(acc_v, out_hbm.at[pl.ds(wbase + r0, CHUNK)])


@jax.jit
def _run(x, tables):
    tab_flat = tables.reshape(N_CB * CB_SIZE, D)
    mesh = plsc.VectorSubcoreMesh(core_axis_name="c", subcore_axis_name="s")
    call = pl.kernel(
        _body,
        out_type=jax.ShapeDtypeStruct((B * T, D), jnp.float32),
        mesh=mesh,
        scratch_types=[
            pltpu.VMEM((N_CB, ROWS_PER_W), jnp.int32),
            pltpu.VMEM((CHUNK, D), jnp.float32),
            pltpu.VMEM((CHUNK, D), jnp.float32),
            pltpu.VMEM((CHUNK, D), jnp.float32),
            pltpu.SemaphoreType.DMA,
            pltpu.SemaphoreType.DMA,
            pltpu.SemaphoreType.DMA,
        ],
    )
    out_flat = call(x, tab_flat)
    return out_flat.reshape(B, T, D)


def kernel(x, tables):
    return _run(x.astype(jnp.int32), tables)


# speedup vs baseline: 2.8063x; 1.1437x over previous
import functools

import jax
import jax.numpy as jnp
from jax import lax
from jax.experimental import pallas as pl
from jax.experimental.pallas import tpu as pltpu
from jax.experimental.pallas import tpu_sc as plsc

N_CB = 8
CB_SIZE = 2048
D = 1024
B = 4
T = 4096

NUM_CORES = 2
NUM_SUBCORES = 16
NUM_WORKERS = NUM_CORES * NUM_SUBCORES
ROWS_PER_W = (B * T) // NUM_WORKERS
CHUNK = 16
N_CHUNKS = ROWS_PER_W // CHUNK
N_PAIRS = N_CHUNKS // 2
VREGS_PER_ROW = D // 16


def _body(x_hbm, tab_hbm, out_hbm, idx_v, acc0_v, acc1_v, bufa_v, bufb_v,
          sa0, sa1, sba, sbb, so0, so1):
    wid = lax.axis_index("s") * NUM_CORES + lax.axis_index("c")
    tpw = T // (NUM_WORKERS // B)
    b = wid // (NUM_WORKERS // B)
    t0 = (wid % (NUM_WORKERS // B)) * tpw
    wbase = wid * ROWS_PER_W

    pltpu.sync_copy(x_hbm.at[b, :, pl.ds(t0, tpw)], idx_v)

    @pl.loop(0, tpw // 16)
    def _offsets(j):
        sl = pl.ds(j * 16, 16)
        for i in range(1, N_CB):
            idx_v[i, sl] = idx_v[i, sl] + i * CB_SIZE

    def gather(cb, r0, dst, sem):
        pltpu.async_copy(tab_hbm.at[idx_v.at[cb, pl.ds(r0, CHUNK)]], dst, sem)

    def wait_gather(dst, sem):
        pltpu.make_async_copy(
            tab_hbm.at[idx_v.at[0, pl.ds(0, CHUNK)]], dst, sem).wait()

    def out_write(acc, r0, sem):
        pltpu.async_copy(acc, out_hbm.at[pl.ds(wbase + r0, CHUNK)], sem)

    def wait_out(acc, sem):
        pltpu.make_async_copy(acc, out_hbm.at[pl.ds(0, CHUNK)], sem).wait()

    def accumulate(acc, buf):
        @plsc.parallel_loop(0, CHUNK * VREGS_PER_ROW, 1, unroll=16)
        def _pairs(j):
            r = j // VREGS_PER_ROW
            c = (j % VREGS_PER_ROW) * 16
            plsc.addupdate(acc.at[r, pl.ds(c, 16)], buf[r, pl.ds(c, 16)])

    gather(0, 0, acc0_v, sa0)
    gather(1, 0, bufa_v, sba)
    gather(2, 0, bufb_v, sbb)

    @pl.loop(0, N_PAIRS)
    def _pair(j):
        r0 = 2 * j * CHUNK
        r1 = r0 + CHUNK
        r_next = r1 + CHUNK
        bufs = (bufa_v, bufb_v)
        bsems = (sba, sbb)
        last = j == N_PAIRS - 1

        wait_gather(acc0_v, sa0)
        for t in range(1, N_CB):
            k = (t - 1) % 2
            wait_gather(bufs[k], bsems[k])
            accumulate(acc0_v, bufs[k])
            if t + 2 < N_CB:
                gather(t + 2, r0, bufs[k], bsems[k])
            elif t + 2 == N_CB:
                gather(1, r1, bufs[k], bsems[k])
            else:
                gather(2, r1, bufs[k], bsems[k])
            if t == 2:
                @pl.when(j > 0)
                def _drain_prev_out():
                    wait_out(acc1_v, so1)
                gather(0, r1, acc1_v, sa1)
        out_write(acc0_v, r0, so0)

        wait_gather(acc1_v, sa1)
        for t in range(1, N_CB):
            k = t % 2
            wait_gather(bufs[k], bsems[k])
            accumulate(acc1_v, bufs[k])
            if t + 2 < N_CB:
                gather(t + 2, r1, bufs[k], bsems[k])
            elif t + 2 >= N_CB:
                @pl.when(jnp.logical_not(last))
                def _prefetch_next():
                    gather(t + 2 - N_CB + 1, r_next, bufs[k], bsems[k])
            if t == 4:
                wait_out(acc0_v, so0)

                @pl.when(jnp.logical_not(last))
                def _prefetch_acc():
                    gather(0, r_next, acc0_v, sa0)
        out_write(acc1_v, r1, so1)

    wait_out(acc1_v, so1)


@jax.jit
def _run(x, tables):
    tab_flat = tables.reshape(N_CB * CB_SIZE, D)
    mesh = plsc.VectorSubcoreMesh(core_axis_name="c", subcore_axis_name="s")
    call = pl.kernel(
        _body,
        out_type=jax.ShapeDtypeStruct((B * T, D), jnp.float32),
        mesh=mesh,
        scratch_types=[
            pltpu.VMEM((N_CB, ROWS_PER_W), jnp.int32),
            pltpu.VMEM((CHUNK, D), jnp.float32),
            pltpu.VMEM((CHUNK, D), jnp.float32),
            pltpu.VMEM((CHUNK, D), jnp.float32),
            pltpu.VMEM((CHUNK, D), jnp.float32),
            pltpu.SemaphoreType.DMA,
            pltpu.SemaphoreType.DMA,
            pltpu.SemaphoreType.DMA,
            pltpu.SemaphoreType.DMA,
            pltpu.SemaphoreType.DMA,
            pltpu.SemaphoreType.DMA,
        ],
    )
    out_flat = call(x, tab_flat)
    return out_flat.reshape(B, T, D)


def kernel(x, tables):
    return _run(x.astype(jnp.int32), tables)
